# baseline (speedup 1.0000x reference)
import functools

import jax
import jax.numpy as jnp
from jax import lax
from jax.experimental import pallas as pl
from jax.experimental.pallas import tpu as pltpu
from jax.experimental.pallas import tpu_sc as plsc

D_MODEL = 64
SCALE = 8.0
LANES = 16

NUM_CORES = 2
NUM_SUBCORES = 16
NUM_WORKERS = NUM_CORES * NUM_SUBCORES

CHUNK = 128
DEPTH = 4
NOUT = 2


def _make_sc_embed(batch: int):
  assert batch % (NUM_WORKERS * CHUNK * DEPTH) == 0
  b_per_w = batch // NUM_WORKERS
  n_chunks = b_per_w // CHUNK
  n_outer = n_chunks // DEPTH

  mesh = plsc.VectorSubcoreMesh(
      core_axis_name="c", subcore_axis_name="s",
      num_cores=NUM_CORES, num_subcores=NUM_SUBCORES)

  @functools.partial(
      pl.kernel,
      mesh=mesh,
      compiler_params=pltpu.CompilerParams(use_tc_tiling_on_sc=False),
      out_type=jax.ShapeDtypeStruct((batch, D_MODEL), jnp.float32),
      scratch_types=[
          pltpu.VMEM((n_chunks, CHUNK), jnp.int32),
          [pltpu.VMEM((CHUNK, D_MODEL), jnp.float32)] * DEPTH,
          [pltpu.VMEM((CHUNK, D_MODEL), jnp.float32)] * NOUT,
          [pltpu.SemaphoreType.DMA] * DEPTH,
          [pltpu.SemaphoreType.DMA] * NOUT,
      ],
  )
  def embed(idx_hbm, table_hbm, out_hbm, idx_v, bufs_in, bufs_out,
            gsems, osems):
    wid = lax.axis_index("s") * NUM_CORES + lax.axis_index("c")
    base = wid * b_per_w

    pltpu.sync_copy(idx_hbm.at[pl.ds(wid * n_chunks, n_chunks)], idx_v)

    def issue_gather(g, b):
      def group16(k, _):
        r0 = k * LANES
        idxv = idx_v[g, pl.ds(r0, LANES)]
        for l in range(LANES):
          row = idxv[l]
          pltpu.async_copy(
              table_hbm.at[pl.ds(row, 1)],
              bufs_in[b].at[pl.ds(r0 + l, 1)], gsems[b])
        return _

      lax.fori_loop(0, CHUNK // LANES, group16, None)

    def wait_gather(b):
      pltpu.make_async_copy(
          table_hbm.at[pl.ds(0, CHUNK)], bufs_in[b], gsems[b]).wait()

    def issue_out(g, o):
      pltpu.async_copy(
          bufs_out[o], out_hbm.at[pl.ds(base + g * CHUNK, CHUNK)], osems[o])

    def wait_out(o):
      pltpu.make_async_copy(
          bufs_out[o], out_hbm.at[pl.ds(0, CHUNK)], osems[o]).wait()

    def scale(b, o):
      src, dst = bufs_in[b], bufs_out[o]

      def rows4(r4, _):
        r = r4 * 4
        for dr in range(4):
          for j in range(D_MODEL // LANES):
            sl = pl.ds(j * LANES, LANES)
            dst[r + dr, sl] = src[r + dr, sl] * SCALE
        return _

      lax.fori_loop(0, CHUNK // 4, rows4, None)

    for b in range(DEPTH):
      issue_gather(b, b)

    def outer(t, _):
      for b in range(DEPTH):
        g = t * DEPTH + b
        o = b % NOUT
        wait_gather(b)
        if b < NOUT:
          @pl.when(t > 0)
          def _wait():
            wait_out(o)
        else:
          wait_out(o)
        scale(b, o)
        issue_out(g, o)

        @pl.when(t < n_outer - 1)
        def _next():
          issue_gather(g + DEPTH, b)
      return _

    lax.fori_loop(0, n_outer, outer, None)

    for o in range(NOUT):
      wait_out(o)

  return embed


def kernel(x, table):
  batch = x.shape[0] * x.shape[1]
  flat_idx = x.reshape(batch // CHUNK, CHUNK).astype(jnp.int32)
  out = _make_sc_embed(batch)(flat_idx, table)
  return out.reshape(x.shape[0], x.shape[1], D_MODEL)
